# Initial kernel scaffold; baseline (speedup 1.0000x reference)
import jax
import jax.numpy as jnp
from jax.experimental import pallas as pl


def kernel(x, Wr, br, W1, b1, W2, b2, W3, b3):
    raise NotImplementedError("write your pallas kernel here")


# speedup vs baseline: 1.0087x; 1.0087x over previous
import functools

import jax
import jax.numpy as jnp
from jax.experimental import pallas as pl
from jax.experimental.pallas import tpu as pltpu

T = 2048
D = 1024
H = 1024
O = 1024
E = 8
K = 2

BLK = 256
NB = (T * K) // BLK + E
NP = NB * BLK


def _gmm_body(be_ref, x_ref, g_ref, w1_ref, b1_ref, w2_ref, b2_ref, w3_ref,
              b3_ref, o_ref):
    x = x_ref[...]
    h1 = jnp.dot(x, w1_ref[0], preferred_element_type=jnp.float32)
    h1 = jnp.maximum(h1 + b1_ref[0], 0.0)
    h2 = jnp.dot(h1, w2_ref[0], preferred_element_type=jnp.float32)
    h2 = jnp.maximum(h2 + b2_ref[0], 0.0)
    eo = jnp.dot(h2, w3_ref[0], preferred_element_type=jnp.float32)
    eo = eo + b3_ref[0]
    o_ref[...] = eo * g_ref[:, 0:1]


def _gmm(block_expert, x_sorted, gates_mat, W1, b1, W2, b2, W3, b3):
    def rows_map(i, be):
        return (i, 0)

    def w_map(i, be):
        return (be[i], 0, 0)

    def b_map(i, be):
        return (be[i], 0, 0)

    grid_spec = pltpu.PrefetchScalarGridSpec(
        num_scalar_prefetch=1,
        grid=(NB,),
        in_specs=[
            pl.BlockSpec((BLK, D), rows_map),
            pl.BlockSpec((BLK, 128), rows_map),
            pl.BlockSpec((1, D, H), w_map),
            pl.BlockSpec((1, 1, H), b_map),
            pl.BlockSpec((1, H, H), w_map),
            pl.BlockSpec((1, 1, H), b_map),
            pl.BlockSpec((1, H, O), w_map),
            pl.BlockSpec((1, 1, O), b_map),
        ],
        out_specs=pl.BlockSpec((BLK, O), rows_map),
    )
    return pl.pallas_call(
        _gmm_body,
        grid_spec=grid_spec,
        out_shape=jax.ShapeDtypeStruct((NP, O), jnp.float32),
    )(block_expert, x_sorted, gates_mat,
      W1, b1.reshape(E, 1, H), W2, b2.reshape(E, 1, H), W3, b3.reshape(E, 1, O))


def kernel(x, Wr, br, W1, b1, W2, b2, W3, b3):
    logits = x @ Wr + br
    probs = jax.nn.softmax(logits, axis=-1)
    w, ids = jax.lax.top_k(probs, K)
    w = w / (jnp.sum(w, axis=-1, keepdims=True) + 1e-6)

    A = T * K
    e_flat = ids.reshape(A)
    w_flat = w.reshape(A)
    oh = (e_flat[:, None] == jnp.arange(E)[None, :]).astype(jnp.int32)
    within = jnp.cumsum(oh, axis=0) - oh
    rank = jnp.take_along_axis(within, e_flat[:, None], axis=1)[:, 0]
    counts = jnp.sum(oh, axis=0)
    nblocks_e = (counts + BLK - 1) // BLK
    cum_blocks = jnp.cumsum(nblocks_e)
    offs_e = jnp.concatenate(
        [jnp.zeros((1,), jnp.int32), cum_blocks[:-1].astype(jnp.int32)]) * BLK
    pos = offs_e[e_flat] + rank
    row_token = jnp.zeros((NP,), jnp.int32).at[pos].set(
        (jnp.arange(A) // K).astype(jnp.int32))
    gates_sorted = jnp.zeros((NP,), jnp.float32).at[pos].set(w_flat)
    bidx = jnp.arange(NB)
    block_expert = jnp.where(
        bidx < cum_blocks[-1],
        jnp.searchsorted(cum_blocks, bidx, side="right"), 0).astype(jnp.int32)

    x_sorted = jnp.take(x, row_token, axis=0)
    gates_mat = jnp.broadcast_to(gates_sorted[:, None], (NP, 128))

    eo_sorted = _gmm(block_expert, x_sorted, gates_mat, W1, b1, W2, b2, W3, b3)

    pos_tk = pos.reshape(T, K)
    out = (jnp.take(eo_sorted, pos_tk[:, 0], axis=0)
           + jnp.take(eo_sorted, pos_tk[:, 1], axis=0))
    return out
